# Initial kernel scaffold; baseline (speedup 1.0000x reference)
import jax
import jax.numpy as jnp
from jax.experimental import pallas as pl


def kernel(x_e, x_r, edge_index, edge_type, w_loop, w_fwd, w_bwd, self_loop, w_rel, bias, gamma, beta):
    raise NotImplementedError("write your pallas kernel here")


# speedup vs baseline: 8.5513x; 8.5513x over previous
import dataclasses
import functools

import jax
import jax.numpy as jnp
from jax import lax
from jax.experimental import pallas as pl
from jax.experimental.pallas import tpu as pltpu
from jax.experimental.pallas import tpu_sc as plsc

_N = 10000
_E = 320000
_D = 128
_R = 100

_NC = 2
_NS = 16
_NW = _NC * _NS

_N_PAD = 10112
_ROWS_PER_SUB = _N_PAD // _NS
_BLK = 128
_NBLK = 157
_EPC = _NBLK * _BLK
_E_PAD = _NS * _EPC
_EPW = _E_PAD // _NW
_XR_ROWS = 2 * _R + 8

_mesh = plsc.VectorSubcoreMesh(core_axis_name="c", subcore_axis_name="s")

_sc_params = pltpu.CompilerParams()
if "needs_layout_passes" in pltpu.CompilerParams.__dataclass_fields__:
    _sc_params = dataclasses.replace(_sc_params, needs_layout_passes=False)


@functools.partial(
    pl.kernel,
    out_type=jax.ShapeDtypeStruct((_NW, 2, _N_PAD), jnp.float32),
    mesh=_mesh,
    compiler_params=_sc_params,
    scratch_types=[
        pltpu.VMEM((_N_PAD,), jnp.float32),
        pltpu.VMEM((_N_PAD,), jnp.float32),
        pltpu.VMEM((_EPW,), jnp.int32),
        pltpu.VMEM((_EPW,), jnp.int32),
    ],
)
def _sc_hist(src_hbm, tgt_hbm, out_hbm, ho, hi, sbuf, tbuf):
    cid = lax.axis_index("c")
    sid = lax.axis_index("s")
    wid = cid * _NS + sid
    zeros = jnp.zeros((16,), jnp.float32)

    @pl.loop(0, _N_PAD, step=16)
    def _(i):
        ho[pl.ds(i, 16)] = zeros
        hi[pl.ds(i, 16)] = zeros

    base = wid * _EPW
    pltpu.sync_copy(src_hbm.at[pl.ds(base, _EPW)], sbuf)
    pltpu.sync_copy(tgt_hbm.at[pl.ds(base, _EPW)], tbuf)
    ones = jnp.ones((16,), jnp.float32)

    @pl.loop(0, _EPW, step=16)
    def _(i):
        plsc.addupdate_scatter(ho, [sbuf[pl.ds(i, 16)]], ones)
        plsc.addupdate_scatter(hi, [tbuf[pl.ds(i, 16)]], ones)

    pltpu.sync_copy(ho, out_hbm.at[wid, 0])
    pltpu.sync_copy(hi, out_hbm.at[wid, 1])


def _tc_prescale_body(hist_ref, xe_ref, xs_ref, xt_ref, uvt_ref):
    deg = jnp.sum(hist_ref[...], axis=0)
    uvt = lax.rsqrt(jnp.maximum(deg, 1.0)).T
    uvt_ref[...] = uvt
    xe = xe_ref[...]
    xs_ref[...] = xe * uvt[:, 0:1]
    xt_ref[...] = xe * uvt[:, 1:2]


_tc_prescale = pl.pallas_call(
    _tc_prescale_body,
    out_shape=(
        jax.ShapeDtypeStruct((_N_PAD, _D), jnp.float32),
        jax.ShapeDtypeStruct((_N_PAD, _D), jnp.float32),
        jax.ShapeDtypeStruct((_N_PAD, 2), jnp.float32),
    ),
)


@functools.partial(
    pl.kernel,
    out_type=(
        jax.ShapeDtypeStruct((_N_PAD, _D), jnp.float32),
        jax.ShapeDtypeStruct((_N_PAD, _D), jnp.float32),
    ),
    mesh=_mesh,
    compiler_params=_sc_params,
    scratch_types=[
        pltpu.VMEM((_BLK,), jnp.int32),
        pltpu.VMEM((_BLK,), jnp.int32),
        pltpu.VMEM((_BLK,), jnp.int32),
        pltpu.VMEM((_BLK,), jnp.int32),
        pltpu.VMEM((_BLK, _D), jnp.float32),
        pltpu.VMEM((_BLK, _D), jnp.float32),
        pltpu.VMEM((128, _D), jnp.float32),
        pltpu.VMEM_SHARED((_N_PAD, _D), jnp.float32),
        pltpu.SemaphoreType.DMA,
        pltpu.SemaphoreType.DMA,
    ],
)
def _sc_agg(xs_hbm, xt_hbm, xr_hbm, src_hbm, tgt_hbm, et_hbm,
            aggf_hbm, aggb_hbm,
            gidx, sidx, etb, xidx, rowb, xrb, zb, acc, sem1, sem2):
    cid = lax.axis_index("c")
    sid = lax.axis_index("s")
    wid = cid * _NS + sid
    zeros = jnp.zeros((16,), jnp.float32)

    @pl.loop(0, 128)
    def _(r):
        @pl.loop(0, _D, step=16)
        def _(c):
            zb[r, pl.ds(c, 16)] = zeros

    row0 = sid * _ROWS_PER_SUB

    @pl.loop(0, 4)
    def _(j):
        pltpu.sync_copy(zb, acc.at[pl.ds(row0 + j * 128, 128)])

    pltpu.sync_copy(zb.at[pl.ds(0, _ROWS_PER_SUB - 512)],
                    acc.at[pl.ds(row0 + 512, _ROWS_PER_SUB - 512)])
    plsc.subcore_barrier()

    def run_dir(tbl_hbm, gsrc_hbm, ssrc_hbm, xr_add, out_hbm):
        woff = wid * _XR_ROWS + xr_add

        @pl.loop(0, _NBLK)
        def _(b):
            ebase = sid * _EPC + b * _BLK
            pltpu.sync_copy(gsrc_hbm.at[pl.ds(ebase, _BLK)], gidx)
            pltpu.sync_copy(ssrc_hbm.at[pl.ds(ebase, _BLK)], sidx)
            pltpu.sync_copy(et_hbm.at[pl.ds(ebase, _BLK)], etb)

            @pl.loop(0, _BLK, step=16)
            def _(i):
                xidx[pl.ds(i, 16)] = etb[pl.ds(i, 16)] * 2 + woff

            cg = pltpu.async_copy(tbl_hbm.at[gidx], rowb, sem1)
            cx = pltpu.async_copy(xr_hbm.at[xidx], xrb, sem2)
            cg.wait()
            cx.wait()

            @pl.loop(0, _BLK)
            def _(r):
                @pl.loop(0, _D, step=16)
                def _(c):
                    rowb[r, pl.ds(c, 16)] = (rowb[r, pl.ds(c, 16)]
                                             * xrb[r, pl.ds(c, 16)])

            pltpu.sync_copy(rowb, acc.at[sidx], add=True)

        plsc.subcore_barrier()

        @pl.loop(0, 4)
        def _(j):
            pltpu.sync_copy(acc.at[pl.ds(row0 + j * 128, 128)],
                            out_hbm.at[pl.ds(row0 + j * 128, 128)])

        pltpu.sync_copy(acc.at[pl.ds(row0 + 512, _ROWS_PER_SUB - 512)],
                        out_hbm.at[pl.ds(row0 + 512, _ROWS_PER_SUB - 512)])

    @pl.when(cid == 0)
    def _():
        run_dir(xs_hbm, src_hbm, tgt_hbm, 0, aggf_hbm)

    @pl.when(cid == 1)
    def _():
        run_dir(xt_hbm, tgt_hbm, src_hbm, 1, aggb_hbm)


def _dot(a, b, dims):
    return lax.dot_general(a, b, (dims, ((), ())),
                           precision=lax.Precision.HIGHEST,
                           preferred_element_type=jnp.float32)


_NB = 2000
_NGRID = _N // _NB


def _tc_mm_body(xe_ref, aggf_ref, aggb_ref, uv_ref, sl_ref, wl_ref,
                wf_ref, wb_ref, bias_ref, hpre_ref, sum_ref):
    i = pl.program_id(0)
    af = aggf_ref[...] * uv_ref[:, 1:2]
    ab = aggb_ref[...] * uv_ref[:, 0:1]
    xe = xe_ref[...]
    h = (_dot(xe * sl_ref[...], wl_ref[...], ((1,), (0,)))
         + _dot(af, wf_ref[...], ((1,), (0,)))
         + _dot(ab, wb_ref[...], ((1,), (0,)))) * (1.0 / 3.0) \
        + bias_ref[...][None, :]
    hpre_ref[...] = h

    @pl.when(i == 0)
    def _():
        sum_ref[...] = jnp.zeros_like(sum_ref)

    sum_ref[0, :] += jnp.sum(h, axis=0)


_tc_mm = pl.pallas_call(
    _tc_mm_body,
    grid=(_NGRID,),
    in_specs=[
        pl.BlockSpec((_NB, _D), lambda i: (i, 0)),
        pl.BlockSpec((_NB, _D), lambda i: (i, 0)),
        pl.BlockSpec((_NB, _D), lambda i: (i, 0)),
        pl.BlockSpec((_NB, 2), lambda i: (i, 0)),
        pl.BlockSpec((1, _D), lambda i: (0, 0)),
        pl.BlockSpec((_D, _D), lambda i: (0, 0)),
        pl.BlockSpec((_D, _D), lambda i: (0, 0)),
        pl.BlockSpec((_D, _D), lambda i: (0, 0)),
        pl.BlockSpec((_D,), lambda i: (0,)),
    ],
    out_specs=(
        pl.BlockSpec((_NB, _D), lambda i: (i, 0)),
        pl.BlockSpec((8, _D), lambda i: (0, 0)),
    ),
    out_shape=(
        jax.ShapeDtypeStruct((_N, _D), jnp.float32),
        jax.ShapeDtypeStruct((8, _D), jnp.float32),
    ),
)


def _tc_norm_body(hpre_ref, sum_ref, gamma_ref, beta_ref, xr_ref, wr_ref,
                  h_ref, xrn_ref):
    mean = sum_ref[0, :] * (1.0 / _N)
    ctr = hpre_ref[...] - mean[None, :]
    var = jnp.mean(ctr * ctr, axis=0)
    h_ref[...] = ctr * (lax.rsqrt(var + 1e-5) * gamma_ref[...])[None, :] \
        + beta_ref[...][None, :]
    xrn_ref[...] = _dot(xr_ref[...], wr_ref[...], ((1,), (1,)))


_tc_norm = pl.pallas_call(
    _tc_norm_body,
    out_shape=(
        jax.ShapeDtypeStruct((_N, _D), jnp.float32),
        jax.ShapeDtypeStruct((2 * _R, _D), jnp.float32),
    ),
)


def kernel(x_e, x_r, edge_index, edge_type, w_loop, w_fwd, w_bwd, self_loop,
           w_rel, bias, gamma, beta):
    pad = _E_PAD - _E
    pad_rows = (_N + (jnp.arange(pad, dtype=jnp.int32) % 8)).astype(jnp.int32)
    src_p = jnp.concatenate([edge_index[0], pad_rows])
    tgt_p = jnp.concatenate([edge_index[1], pad_rows])
    et_p = jnp.concatenate(
        [edge_type.astype(jnp.int32), jnp.full((pad,), _R, jnp.int32)])
    xe_p = jnp.pad(x_e, ((0, _N_PAD - _N), (0, 0)))
    xr_rep = jnp.tile(jnp.pad(x_r, ((0, 8), (0, 0))), (_NW, 1))

    hist = _sc_hist(src_p, tgt_p)
    xs, xt, uv = _tc_prescale(hist, xe_p)
    aggf, aggb = _sc_agg(xs, xt, xr_rep, src_p, tgt_p, et_p)
    hpre, colsum = _tc_mm(x_e, aggf, aggb, uv, self_loop, w_loop, w_fwd,
                          w_bwd, bias)
    h, xrn = _tc_norm(hpre, colsum, gamma, beta, x_r, w_rel)
    return (h, xrn)
